# baseline (speedup 1.0000x reference)
import functools

import jax
import jax.numpy as jnp
from jax import lax
from jax.experimental import pallas as pl
from jax.experimental.pallas import tpu as pltpu
from jax.experimental.pallas import tpu_sc as plsc

NC = 2
NS = 16
NW = NC * NS
L = 16

_MESH = dict(core_axis_name="c", subcore_axis_name="s", num_cores=NC)


def _gather_call(V, D, J, B, NBUF, GD):
    CH = B // NW
    n_ch = J
    WD = NBUF - GD
    assert 1 <= GD < NBUF and n_ch % NBUF == 0
    n_outer = n_ch // NBUF
    assert n_outer >= 2

    scratch = (
        [pltpu.VMEM((CH,), jnp.int32) for _ in range(NBUF)]
        + [pltpu.VMEM((CH, D), jnp.float32) for _ in range(NBUF)]
        + [pltpu.SemaphoreType.DMA for _ in range(2 * NBUF)]
    )

    @functools.partial(
        pl.kernel,
        out_type=jax.ShapeDtypeStruct((J * B, D), jnp.float32),
        mesh=plsc.VectorSubcoreMesh(**_MESH),
        scratch_types=scratch,
        compiler_params=pltpu.CompilerParams(use_tc_tiling_on_sc=False),
    )
    def gather_kernel(table_hbm, idx_hbm, out_hbm, *refs):
        idx_v = refs[0:NBUF]
        rows_v = refs[NBUF:2 * NBUF]
        sem_g = refs[2 * NBUF:3 * NBUF]
        sem_w = refs[3 * NBUF:4 * NBUF]
        wid = lax.axis_index("s") * NC + lax.axis_index("c")
        boff = wid * CH

        def start_gather(c, b):
            pltpu.sync_copy(idx_hbm.at[c, pl.ds(boff, CH)], idx_v[b])
            pltpu.async_copy(table_hbm.at[idx_v[b]], rows_v[b], sem_g[b])

        def wait_gather(b):
            pltpu.make_async_copy(table_hbm.at[idx_v[b]], rows_v[b],
                                  sem_g[b]).wait()

        def start_wb(c, b):
            pltpu.async_copy(rows_v[b], out_hbm.at[pl.ds(c * B + boff, CH)],
                             sem_w[b])

        def wait_wb(b):
            pltpu.make_async_copy(rows_v[b], out_hbm.at[pl.ds(boff, CH)],
                                  sem_w[b]).wait()

        def step(r, b, do_input, do_waitwb):
            ib = (b + GD) % NBUF
            if do_input:
                if do_waitwb:
                    wait_wb(ib)
                start_gather(r + GD, ib)
            wait_gather(b)
            start_wb(r, b)

        for c in range(GD):
            start_gather(c, c)
        for b in range(NBUF):
            step(b, b, True, b >= WD)

        def outer(o, carry):
            r0 = o * NBUF
            for b in range(NBUF):
                step(r0 + b, b, True, True)
            return carry

        lax.fori_loop(1, n_outer - 1, outer, 0)

        r0 = (n_outer - 1) * NBUF
        for b in range(NBUF):
            step(r0 + b, b, b < WD, True)
        for b in range(NBUF):
            wait_wb(b)

    return gather_kernel


def kernel(weight, nodes):
    V, D = weight.shape
    Bt, J = nodes.shape
    nodes_t = jnp.transpose(nodes)
    out2d = _gather_call(V, D, J, Bt, NBUF=5, GD=2)(weight, nodes_t)
    out_t = out2d.reshape(J, Bt, D)
    return jnp.transpose(out_t, (1, 0, 2))
